# baseline (speedup 1.0000x reference)
import jax
import jax.numpy as jnp
from jax.experimental import pallas as pl
from jax.experimental.pallas import tpu as pltpu

_EPS = 1e-5
_ROWS = 200


def _bn_relu(y, g, b):
    return jnp.maximum(g * (y * (1.0 / jnp.sqrt(1.0 + _EPS))) + b, 0.0)


def _fused_body(nblk, x_ref, adj_ref, w1_ref, b1_ref, g_ref, be_ref,
                w2_ref, b2_ref, m1w_ref, m1b_ref, m1g_ref, m1be_ref,
                m2w_ref, m2b_ref, m2g_ref, m2be_ref, m3w_ref, m3b_ref,
                out_ref, s1_ref, s2_ref):
    i = pl.program_id(0)

    @pl.when(i == 0)
    def _support():
        s1_ref[...] = jnp.dot(x_ref[...], w1_ref[...],
                              preferred_element_type=jnp.float32)

    @pl.when((i >= 1) & (i <= nblk))
    def _pass1():
        y = jnp.dot(adj_ref[...], s1_ref[...],
                    preferred_element_type=jnp.float32)
        x1 = _bn_relu(y + b1_ref[...], g_ref[...], be_ref[...])
        s2_ref[pl.ds((i - 1) * _ROWS, _ROWS), :] = jnp.dot(
            x1, w2_ref[...], preferred_element_type=jnp.float32)

    @pl.when(i > nblk)
    def _pass2():
        y = jnp.dot(adj_ref[...], s2_ref[...],
                    preferred_element_type=jnp.float32)
        x2 = _bn_relu(y + b2_ref[...], g_ref[...], be_ref[...])
        h = _bn_relu(jnp.dot(x2, m1w_ref[...],
                             preferred_element_type=jnp.float32)
                     + m1b_ref[...], m1g_ref[...], m1be_ref[...])
        h = _bn_relu(jnp.dot(h, m2w_ref[...],
                             preferred_element_type=jnp.float32)
                     + m2b_ref[...], m2g_ref[...], m2be_ref[...])
        o = jnp.dot(h, m3w_ref[...],
                    preferred_element_type=jnp.float32) + m3b_ref[...]
        m = jnp.max(o, axis=1, keepdims=True)
        lse = jnp.log(jnp.sum(jnp.exp(o - m), axis=1, keepdims=True)) + m
        out_ref[...] = o - lse


def _const_spec(shape):
    return pl.BlockSpec(shape, lambda i: (0,) * len(shape))


def kernel(x, adj, W1, b1, W2, b2, bn1_g, bn1_b, m1_W, m1_b, m1_g, m1_be,
           m2_W, m2_b, m2_g, m2_be, m3_W, m3_b):
    n, nfeat = x.shape
    nhid = W1.shape[1]
    nmid = m1_W.shape[1]
    nclass = m3_W.shape[1]
    f32 = jnp.float32
    nblk = n // _ROWS

    def row(v):
        return v.reshape(1, -1)

    def adj_map(i):
        r = jnp.where(i <= nblk, jnp.maximum(i - 1, 0), i - 1 - nblk)
        return (r, 0)

    def out_map(i):
        return (jnp.maximum(i - 1 - nblk, 0), 0)

    import functools
    body = functools.partial(_fused_body, nblk)

    out = pl.pallas_call(
        body,
        grid=(1 + 2 * nblk,),
        in_specs=[_const_spec((n, nfeat)),
                  pl.BlockSpec((_ROWS, n), adj_map),
                  _const_spec((nfeat, nhid)), _const_spec((1, nhid)),
                  _const_spec((1, nhid)), _const_spec((1, nhid)),
                  _const_spec((nhid, nhid)), _const_spec((1, nhid)),
                  _const_spec((nhid, nmid)), _const_spec((1, nmid)),
                  _const_spec((1, nmid)), _const_spec((1, nmid)),
                  _const_spec((nmid, nhid)), _const_spec((1, nhid)),
                  _const_spec((1, nhid)), _const_spec((1, nhid)),
                  _const_spec((nhid, nclass)), _const_spec((1, nclass))],
        out_specs=pl.BlockSpec((_ROWS, nclass), out_map),
        out_shape=jax.ShapeDtypeStruct((n, nclass), f32),
        scratch_shapes=[pltpu.VMEM((n, nhid), f32),
                        pltpu.VMEM((n, nhid), f32)],
        compiler_params=pltpu.CompilerParams(
            dimension_semantics=("arbitrary",)),
    )(x, adj, W1, row(b1), row(bn1_g), row(bn1_b), W2, row(b2),
      m1_W, row(m1_b), row(m1_g), row(m1_be),
      m2_W, row(m2_b), row(m2_g), row(m2_be),
      m3_W, row(m3_b))
    return out


# speedup vs baseline: 1.0692x; 1.0692x over previous
import jax
import jax.numpy as jnp
from jax.experimental import pallas as pl
from jax.experimental.pallas import tpu as pltpu

_EPS = 1e-5
_ROWS = 400


def _bn_relu(y, g, b):
    return jnp.maximum(g * (y * (1.0 / jnp.sqrt(1.0 + _EPS))) + b, 0.0)


def _fused_body(nblk, x_ref, adj_ref, w1_ref, b1_ref, g_ref, be_ref,
                w2_ref, b2_ref, m1w_ref, m1b_ref, m1g_ref, m1be_ref,
                m2w_ref, m2b_ref, m2g_ref, m2be_ref, m3w_ref, m3b_ref,
                out_ref, s1_ref, s2_ref):
    i = pl.program_id(0)

    @pl.when(i == 0)
    def _support():
        s1_ref[...] = jnp.dot(x_ref[...], w1_ref[...],
                              preferred_element_type=jnp.float32)

    @pl.when((i >= 1) & (i <= nblk))
    def _pass1():
        y = jnp.dot(adj_ref[...], s1_ref[...],
                    preferred_element_type=jnp.float32)
        x1 = _bn_relu(y + b1_ref[...], g_ref[...], be_ref[...])
        s2_ref[pl.ds((i - 1) * _ROWS, _ROWS), :] = jnp.dot(
            x1, w2_ref[...], preferred_element_type=jnp.float32)

    @pl.when(i > nblk)
    def _pass2():
        y = jnp.dot(adj_ref[...], s2_ref[...],
                    preferred_element_type=jnp.float32)
        x2 = _bn_relu(y + b2_ref[...], g_ref[...], be_ref[...])
        h = _bn_relu(jnp.dot(x2, m1w_ref[...],
                             preferred_element_type=jnp.float32)
                     + m1b_ref[...], m1g_ref[...], m1be_ref[...])
        h = _bn_relu(jnp.dot(h, m2w_ref[...],
                             preferred_element_type=jnp.float32)
                     + m2b_ref[...], m2g_ref[...], m2be_ref[...])
        o = jnp.dot(h, m3w_ref[...],
                    preferred_element_type=jnp.float32) + m3b_ref[...]
        m = jnp.max(o, axis=1, keepdims=True)
        lse = jnp.log(jnp.sum(jnp.exp(o - m), axis=1, keepdims=True)) + m
        out_ref[...] = o - lse


def _const_spec(shape):
    return pl.BlockSpec(shape, lambda i: (0,) * len(shape))


def kernel(x, adj, W1, b1, W2, b2, bn1_g, bn1_b, m1_W, m1_b, m1_g, m1_be,
           m2_W, m2_b, m2_g, m2_be, m3_W, m3_b):
    n, nfeat = x.shape
    nhid = W1.shape[1]
    nmid = m1_W.shape[1]
    nclass = m3_W.shape[1]
    f32 = jnp.float32
    nblk = n // _ROWS

    def row(v):
        return v.reshape(1, -1)

    def adj_map(i):
        r = jnp.where(i <= nblk, jnp.maximum(i - 1, 0), i - 1 - nblk)
        return (r, 0)

    def out_map(i):
        return (jnp.maximum(i - 1 - nblk, 0), 0)

    import functools
    body = functools.partial(_fused_body, nblk)

    out = pl.pallas_call(
        body,
        grid=(1 + 2 * nblk,),
        in_specs=[_const_spec((n, nfeat)),
                  pl.BlockSpec((_ROWS, n), adj_map),
                  _const_spec((nfeat, nhid)), _const_spec((1, nhid)),
                  _const_spec((1, nhid)), _const_spec((1, nhid)),
                  _const_spec((nhid, nhid)), _const_spec((1, nhid)),
                  _const_spec((nhid, nmid)), _const_spec((1, nmid)),
                  _const_spec((1, nmid)), _const_spec((1, nmid)),
                  _const_spec((nmid, nhid)), _const_spec((1, nhid)),
                  _const_spec((1, nhid)), _const_spec((1, nhid)),
                  _const_spec((nhid, nclass)), _const_spec((1, nclass))],
        out_specs=pl.BlockSpec((_ROWS, nclass), out_map),
        out_shape=jax.ShapeDtypeStruct((n, nclass), f32),
        scratch_shapes=[pltpu.VMEM((n, nhid), f32),
                        pltpu.VMEM((n, nhid), f32)],
        compiler_params=pltpu.CompilerParams(
            dimension_semantics=("arbitrary",)),
    )(x, adj, W1, row(b1), row(bn1_g), row(bn1_b), W2, row(b2),
      m1_W, row(m1_b), row(m1_g), row(m1_be),
      m2_W, row(m2_b), row(m2_g), row(m2_be),
      m3_W, row(m3_b))
    return out


# speedup vs baseline: 1.0778x; 1.0081x over previous
import functools

import jax
import jax.numpy as jnp
from jax.experimental import pallas as pl
from jax.experimental.pallas import tpu as pltpu

_EPS = 1e-5
_ROWS = 400


def _bn_relu(y, g, b):
    return jnp.maximum(g * (y * (1.0 / jnp.sqrt(1.0 + _EPS))) + b, 0.0)


def _fused_body(nblk, adj_ref, x_ref, w1_ref, b1_ref, g_ref, be_ref,
                w2_ref, b2_ref, m1w_ref, m1b_ref, m1g_ref, m1be_ref,
                m2w_ref, m2b_ref, m2g_ref, m2be_ref, m3w_ref, m3b_ref,
                out_ref, s2_ref):
    i = pl.program_id(0)

    @pl.when(i < nblk)
    def _pass1():
        s1 = jnp.dot(x_ref[...], w1_ref[...],
                     preferred_element_type=jnp.float32)
        y = jnp.dot(adj_ref[...], s1, preferred_element_type=jnp.float32)
        x1 = _bn_relu(y + b1_ref[...], g_ref[...], be_ref[...])
        s2_ref[pl.ds(i * _ROWS, _ROWS), :] = jnp.dot(
            x1, w2_ref[...], preferred_element_type=jnp.float32)

    @pl.when(i >= nblk)
    def _pass2():
        y = jnp.dot(adj_ref[...], s2_ref[...],
                    preferred_element_type=jnp.float32)
        x2 = _bn_relu(y + b2_ref[...], g_ref[...], be_ref[...])
        h = _bn_relu(jnp.dot(x2, m1w_ref[...],
                             preferred_element_type=jnp.float32)
                     + m1b_ref[...], m1g_ref[...], m1be_ref[...])
        h = _bn_relu(jnp.dot(h, m2w_ref[...],
                             preferred_element_type=jnp.float32)
                     + m2b_ref[...], m2g_ref[...], m2be_ref[...])
        o = jnp.dot(h, m3w_ref[...],
                    preferred_element_type=jnp.float32) + m3b_ref[...]
        m = jnp.max(o, axis=1, keepdims=True)
        lse = jnp.log(jnp.sum(jnp.exp(o - m), axis=1, keepdims=True)) + m
        out_ref[...] = o - lse


def _const_spec(shape):
    return pl.BlockSpec(shape, lambda i: (0,) * len(shape))


def kernel(x, adj, W1, b1, W2, b2, bn1_g, bn1_b, m1_W, m1_b, m1_g, m1_be,
           m2_W, m2_b, m2_g, m2_be, m3_W, m3_b):
    n, nfeat = x.shape
    nhid = W1.shape[1]
    nmid = m1_W.shape[1]
    nclass = m3_W.shape[1]
    f32 = jnp.float32
    nblk = n // _ROWS

    def row(v):
        return v.reshape(1, -1)

    def adj_map(i):
        return (jnp.where(i < nblk, i, i - nblk), 0)

    def out_map(i):
        return (jnp.maximum(i - nblk, 0), 0)

    body = functools.partial(_fused_body, nblk)

    out = pl.pallas_call(
        body,
        grid=(2 * nblk,),
        in_specs=[pl.BlockSpec((_ROWS, n), adj_map),
                  _const_spec((n, nfeat)),
                  _const_spec((nfeat, nhid)), _const_spec((1, nhid)),
                  _const_spec((1, nhid)), _const_spec((1, nhid)),
                  _const_spec((nhid, nhid)), _const_spec((1, nhid)),
                  _const_spec((nhid, nmid)), _const_spec((1, nmid)),
                  _const_spec((1, nmid)), _const_spec((1, nmid)),
                  _const_spec((nmid, nhid)), _const_spec((1, nhid)),
                  _const_spec((1, nhid)), _const_spec((1, nhid)),
                  _const_spec((nhid, nclass)), _const_spec((1, nclass))],
        out_specs=pl.BlockSpec((_ROWS, nclass), out_map),
        out_shape=jax.ShapeDtypeStruct((n, nclass), f32),
        scratch_shapes=[pltpu.VMEM((n, nhid), f32)],
        compiler_params=pltpu.CompilerParams(
            dimension_semantics=("arbitrary",)),
    )(adj, x, W1, row(b1), row(bn1_g), row(bn1_b), W2, row(b2),
      m1_W, row(m1_b), row(m1_g), row(m1_be),
      m2_W, row(m2_b), row(m2_g), row(m2_be),
      m3_W, row(m3_b))
    return out
